# baseline (speedup 1.0000x reference)
import functools

import jax
import jax.numpy as jnp
from jax import lax
from jax.experimental import pallas as pl
from jax.experimental.pallas import tpu as pltpu
from jax.experimental.pallas import tpu_sc as plsc

HIDDEN = 64
GAMMA = 12.0
BATCH = 16384
NC, NS, L = 2, 16, 16
NW = NC * NS
BPW = BATCH // NW
CH = 128
NCHUNK = BPW // CH


def _make_sc_kernel():
    mesh = plsc.VectorSubcoreMesh(
        core_axis_name="c", subcore_axis_name="s",
        num_cores=NC, num_subcores=NS,
    )

    @functools.partial(
        pl.kernel,
        out_type=jax.ShapeDtypeStruct((BATCH,), jnp.float32),
        mesh=mesh,
        compiler_params=pltpu.CompilerParams(
            needs_layout_passes=False,
            use_tc_tiling_on_sc=False,
        ),
        scratch_types=[
            pltpu.VMEM((BPW,), jnp.int32),
            pltpu.VMEM((BPW,), jnp.int32),
            pltpu.VMEM((BPW,), jnp.int32),
            pltpu.VMEM((BPW,), jnp.int32),
            pltpu.VMEM((BPW,), jnp.int32),
            pltpu.VMEM((BPW,), jnp.int32),
            pltpu.VMEM((2, CH, 2 * HIDDEN), jnp.float32),
            pltpu.VMEM((2, CH, 2 * HIDDEN), jnp.float32),
            pltpu.VMEM((2, CH, 2 * HIDDEN), jnp.float32),
            pltpu.VMEM((BPW, L), jnp.float32),
            pltpu.VMEM((BPW,), jnp.float32),
            pltpu.SemaphoreType.DMA,
            pltpu.SemaphoreType.DMA,
        ],
    )
    def ker(hp_hbm, rp_hbm, tp_hbm, hoff_hbm, roff_hbm, toff_hbm,
            ent2_hbm, rel2_hbm, out_hbm,
            hp, rp, tp, hpar, rpar, tpar,
            hrow, rrow, trow, psum, scores, sem0, sem1):
        wid = lax.axis_index("s") * NC + lax.axis_index("c")
        base = wid * BPW

        pltpu.sync_copy(hp_hbm.at[pl.ds(base, BPW)], hp)
        pltpu.sync_copy(rp_hbm.at[pl.ds(base, BPW)], rp)
        pltpu.sync_copy(tp_hbm.at[pl.ds(base, BPW)], tp)
        pltpu.sync_copy(hoff_hbm.at[pl.ds(base, BPW)], hpar)
        pltpu.sync_copy(roff_hbm.at[pl.ds(base, BPW)], rpar)
        pltpu.sync_copy(toff_hbm.at[pl.ds(base, BPW)], tpar)

        sems = (sem0, sem1)

        def fire(c):
            slot = c % 2
            s = sems[slot]
            idx = pl.ds(c * CH, CH)
            return (
                pltpu.async_copy(ent2_hbm.at[hp.at[idx]], hrow.at[slot], s),
                pltpu.async_copy(rel2_hbm.at[rp.at[idx]], rrow.at[slot], s),
                pltpu.async_copy(ent2_hbm.at[tp.at[idx]], trow.at[slot], s),
            )

        pending = {0: fire(0), 1: fire(1)}

        for c in range(NCHUNK):
            for cp in pending.pop(c):
                cp.wait()
            slot = c % 2

            def row_body(j, carry, slot=slot, cbase=c * CH):
                b = cbase + j
                bs = jnp.full((L,), b, jnp.int32)
                mh = plsc.load_gather(hpar, [bs]) != 0
                mr = plsc.load_gather(rpar, [bs]) != 0
                mt = plsc.load_gather(tpar, [bs]) != 0
                p = None
                for k in range(HIDDEN // L):
                    h = jnp.where(mh, hrow[slot, j, pl.ds(HIDDEN + k * L, L)],
                                  hrow[slot, j, pl.ds(k * L, L)])
                    r = jnp.where(mr, rrow[slot, j, pl.ds(HIDDEN + k * L, L)],
                                  rrow[slot, j, pl.ds(k * L, L)])
                    t = jnp.where(mt, trow[slot, j, pl.ds(HIDDEN + k * L, L)],
                                  trow[slot, j, pl.ds(k * L, L)])
                    a = jnp.abs((h + r) - t)
                    p = a if p is None else p + a
                psum[b, :] = plsc.cumsum(p)
                return carry

            lax.fori_loop(0, CH, row_body, 0)
            if c + 2 < NCHUNK:
                pending[c + 2] = fire(c + 2)

        iot = lax.iota(jnp.int32, L)
        c15 = jnp.full((L,), L - 1, jnp.int32)
        for g in range(BPW // L):
            s = plsc.load_gather(psum, [iot + (g * L), c15])
            scores[pl.ds(g * L, L)] = GAMMA - s

        pltpu.sync_copy(scores, out_hbm.at[pl.ds(base, BPW)])

    return ker


TW = 40960


def _tc_transpose_pairs(table_t):
    n = table_t.shape[1]
    grid = (n + TW - 1) // TW
    half = TW // 2

    def body(x_ref, o_ref):
        t = x_ref[...].T
        o_ref[...] = jnp.concatenate(
            [t[:half, :], t[half:, :]], axis=1)

    return pl.pallas_call(
        body,
        grid=(grid,),
        in_specs=[pl.BlockSpec((HIDDEN, TW), lambda j: (0, j))],
        out_specs=pl.BlockSpec((half, 2 * HIDDEN), lambda j: (j, 0)),
        out_shape=jax.ShapeDtypeStruct((grid * half, 2 * HIDDEN), jnp.float32),
        compiler_params=pltpu.CompilerParams(
            vmem_limit_bytes=128 * 1024 * 1024),
    )(table_t)


def kernel(sample, entity_embedding, relation_embedding):
    h_id = sample[:, 0]
    r_id = sample[:, 1]
    t_id = sample[:, 2]
    half = TW // 2

    def pair_idx(e):
        return ((e // TW) * half + (e % half)).reshape(-1)

    def parity(e):
        return ((e % TW) // half).reshape(-1)

    hp, rp, tp = pair_idx(h_id), pair_idx(r_id), pair_idx(t_id)
    hoff, roff, toff = parity(h_id), parity(r_id), parity(t_id)
    ent2 = _tc_transpose_pairs(entity_embedding.T)
    rel2 = _tc_transpose_pairs(relation_embedding.T)
    score = _make_sc_kernel()(hp, rp, tp, hoff, roff, toff, ent2, rel2)
    return score[:, None]


# speedup vs baseline: 1.0188x; 1.0188x over previous
import functools

import jax
import jax.numpy as jnp
from jax import lax
from jax.experimental import pallas as pl
from jax.experimental.pallas import tpu as pltpu
from jax.experimental.pallas import tpu_sc as plsc

HIDDEN = 64
GAMMA = 12.0
BATCH = 16384
NC, NS, L = 2, 16, 16
NW = NC * NS
BPW = BATCH // NW
CH = 128
NCHUNK = BPW // CH


def _make_sc_kernel():
    mesh = plsc.VectorSubcoreMesh(
        core_axis_name="c", subcore_axis_name="s",
        num_cores=NC, num_subcores=NS,
    )

    @functools.partial(
        pl.kernel,
        out_type=jax.ShapeDtypeStruct((BATCH,), jnp.float32),
        mesh=mesh,
        compiler_params=pltpu.CompilerParams(
            needs_layout_passes=False,
            use_tc_tiling_on_sc=False,
        ),
        scratch_types=[
            pltpu.VMEM((BPW,), jnp.int32),
            pltpu.VMEM((BPW,), jnp.int32),
            pltpu.VMEM((BPW,), jnp.int32),
            pltpu.VMEM((BPW,), jnp.int32),
            pltpu.VMEM((BPW,), jnp.int32),
            pltpu.VMEM((BPW,), jnp.int32),
            pltpu.VMEM((2, CH, 2 * HIDDEN), jnp.float32),
            pltpu.VMEM((2, CH, 2 * HIDDEN), jnp.float32),
            pltpu.VMEM((2, CH, 2 * HIDDEN), jnp.float32),
            pltpu.VMEM((BPW, L), jnp.float32),
            pltpu.VMEM((BPW,), jnp.float32),
            pltpu.SemaphoreType.DMA,
            pltpu.SemaphoreType.DMA,
        ],
    )
    def ker(hp_hbm, rp_hbm, tp_hbm, hoff_hbm, roff_hbm, toff_hbm,
            ent2_hbm, rel2_hbm, out_hbm,
            hp, rp, tp, hpar, rpar, tpar,
            hrow, rrow, trow, psum, scores, sem0, sem1):
        wid = lax.axis_index("s") * NC + lax.axis_index("c")
        base = wid * BPW

        pltpu.sync_copy(hp_hbm.at[pl.ds(base, BPW)], hp)
        pltpu.sync_copy(rp_hbm.at[pl.ds(base, BPW)], rp)
        pltpu.sync_copy(tp_hbm.at[pl.ds(base, BPW)], tp)
        pltpu.sync_copy(hoff_hbm.at[pl.ds(base, BPW)], hpar)
        pltpu.sync_copy(roff_hbm.at[pl.ds(base, BPW)], rpar)
        pltpu.sync_copy(toff_hbm.at[pl.ds(base, BPW)], tpar)

        sems = (sem0, sem1)

        def fire(c):
            slot = c % 2
            s = sems[slot]
            idx = pl.ds(c * CH, CH)
            return (
                pltpu.async_copy(ent2_hbm.at[hp.at[idx]], hrow.at[slot], s),
                pltpu.async_copy(rel2_hbm.at[rp.at[idx]], rrow.at[slot], s),
                pltpu.async_copy(ent2_hbm.at[tp.at[idx]], trow.at[slot], s),
            )

        pending = {0: fire(0), 1: fire(1)}

        for c in range(NCHUNK):
            for cp in pending.pop(c):
                cp.wait()
            slot = c % 2

            def row_body(j, carry, slot=slot, cbase=c * CH):
                b = cbase + j
                bs = jnp.full((L,), b, jnp.int32)
                mh = plsc.load_gather(hpar, [bs]) != 0
                mr = plsc.load_gather(rpar, [bs]) != 0
                mt = plsc.load_gather(tpar, [bs]) != 0
                p = None
                for k in range(HIDDEN // L):
                    h = jnp.where(mh, hrow[slot, j, pl.ds(HIDDEN + k * L, L)],
                                  hrow[slot, j, pl.ds(k * L, L)])
                    r = jnp.where(mr, rrow[slot, j, pl.ds(HIDDEN + k * L, L)],
                                  rrow[slot, j, pl.ds(k * L, L)])
                    t = jnp.where(mt, trow[slot, j, pl.ds(HIDDEN + k * L, L)],
                                  trow[slot, j, pl.ds(k * L, L)])
                    a = jnp.abs((h + r) - t)
                    p = a if p is None else p + a
                psum[b, :] = plsc.cumsum(p)
                return carry

            lax.fori_loop(0, CH, row_body, 0)
            if c + 2 < NCHUNK:
                pending[c + 2] = fire(c + 2)

        iot = lax.iota(jnp.int32, L)
        c15 = jnp.full((L,), L - 1, jnp.int32)
        for g in range(BPW // L):
            s = plsc.load_gather(psum, [iot + (g * L), c15])
            scores[pl.ds(g * L, L)] = GAMMA - s

        pltpu.sync_copy(scores, out_hbm.at[pl.ds(base, BPW)])

    return ker


TW = 32768


def _tc_transpose_pairs(table_t):
    n = table_t.shape[1]
    grid = (n + TW - 1) // TW
    half = TW // 2

    def body(x_ref, o_ref):
        t = x_ref[...].T
        o_ref[...] = jnp.concatenate(
            [t[:half, :], t[half:, :]], axis=1)

    return pl.pallas_call(
        body,
        grid=(grid,),
        in_specs=[pl.BlockSpec((HIDDEN, TW), lambda j: (0, j))],
        out_specs=pl.BlockSpec((half, 2 * HIDDEN), lambda j: (j, 0)),
        out_shape=jax.ShapeDtypeStruct((grid * half, 2 * HIDDEN), jnp.float32),
    )(table_t)


def kernel(sample, entity_embedding, relation_embedding):
    h_id = sample[:, 0]
    r_id = sample[:, 1]
    t_id = sample[:, 2]
    half = TW // 2

    def pair_idx(e):
        return ((e // TW) * half + (e % half)).reshape(-1)

    def parity(e):
        return ((e % TW) // half).reshape(-1)

    hp, rp, tp = pair_idx(h_id), pair_idx(r_id), pair_idx(t_id)
    hoff, roff, toff = parity(h_id), parity(r_id), parity(t_id)
    ent2 = _tc_transpose_pairs(entity_embedding.T)
    rel2 = _tc_transpose_pairs(relation_embedding.T)
    score = _make_sc_kernel()(hp, rp, tp, hoff, roff, toff, ent2, rel2)
    return score[:, None]
